# Initial kernel scaffold; baseline (speedup 1.0000x reference)
import jax
import jax.numpy as jnp
from jax.experimental import pallas as pl


def kernel(center_words, pos_context, neg_context, embedding_v, embedding_u):
    raise NotImplementedError("write your pallas kernel here")


# speedup vs baseline: 5.0301x; 5.0301x over previous
import functools

import jax
import jax.numpy as jnp
from jax import lax
from jax.experimental import pallas as pl
from jax.experimental.pallas import tpu as pltpu
from jax.experimental.pallas import tpu_sc as plsc

B = 16384
K_NEG = 20
KP = 32
D = 64
V = 1000000

NC = 2
NS = 16
NW = NC * NS
SPW = B // NW
G = 16
NG = SPW // G
NEG_ROWS_G = G * K_NEG
NEG_IDX_ROWS_G = NEG_ROWS_G // 64
NEG_IDX_ROWS_W = SPW * K_NEG // 64

REPACK_BM = 8192


def _repack_body(v_ref, u_ref, out_ref):
    out_ref[...] = jnp.concatenate((v_ref[...], u_ref[...]), axis=-1)


def _repack(emb_v, emb_u):
    return pl.pallas_call(
        _repack_body,
        grid=(V // REPACK_BM,),
        in_specs=[
            pl.BlockSpec((REPACK_BM, D), lambda i: (i, 0)),
            pl.BlockSpec((REPACK_BM, D), lambda i: (i, 0)),
        ],
        out_specs=pl.BlockSpec((REPACK_BM, 2 * D), lambda i: (i, 0)),
        out_shape=jax.ShapeDtypeStruct((V, 2 * D), jnp.float32),
    )(emb_v, emb_u)


def _sc_logits(center2, pos2, neg2, cat):
    mesh = plsc.VectorSubcoreMesh(core_axis_name="c", subcore_axis_name="s")

    @functools.partial(
        pl.kernel,
        mesh=mesh,
        out_type=[
            jax.ShapeDtypeStruct((B,), jnp.float32),
            jax.ShapeDtypeStruct((B * KP,), jnp.float32),
        ],
        scratch_types=[
            pltpu.VMEM((NG, G), jnp.int32),
            pltpu.VMEM((NG, G), jnp.int32),
            pltpu.VMEM((NEG_IDX_ROWS_W, 64), jnp.int32),
            pltpu.VMEM((2, G, 2 * D), jnp.float32),
            pltpu.VMEM((2, G, 2 * D), jnp.float32),
            pltpu.VMEM((2, NEG_ROWS_G, 2 * D), jnp.float32),
            pltpu.VMEM((SPW,), jnp.float32),
            pltpu.VMEM((G * KP,), jnp.float32),
            pltpu.SemaphoreType.DMA,
            pltpu.SemaphoreType.DMA,
        ],
    )
    def body(center_r, pos_r, neg_r, cat_r, pos_out, neg_out,
             idx_c, idx_p, idx_n, cb, pb, nb, ps, ns, sem0, sem1):
        wid = lax.axis_index("s") * NC + lax.axis_index("c")
        sems = (sem0, sem1)
        lanes = lax.iota(jnp.int32, 16)

        pltpu.sync_copy(center_r.at[pl.ds(wid * NG, NG)], idx_c)
        pltpu.sync_copy(pos_r.at[pl.ds(wid * NG, NG)], idx_p)
        pltpu.sync_copy(neg_r.at[pl.ds(wid * NEG_IDX_ROWS_W, NEG_IDX_ROWS_W)],
                        idx_n)

        def issue(g, par):
            sem = sems[par]
            pltpu.async_copy(cat_r.at[idx_c.at[g]], cb.at[par], sem)
            pltpu.async_copy(cat_r.at[idx_p.at[g]], pb.at[par], sem)
            for j in range(NEG_IDX_ROWS_G):
                pltpu.async_copy(
                    cat_r.at[idx_n.at[g * NEG_IDX_ROWS_G + j]],
                    nb.at[par].at[pl.ds(j * 64, 64)], sem)

        def drain(par):
            sem = sems[par]
            pltpu.make_async_copy(cat_r.at[pl.ds(0, G)], cb.at[par], sem).wait()
            pltpu.make_async_copy(cat_r.at[pl.ds(0, G)], pb.at[par], sem).wait()
            pltpu.make_async_copy(cat_r.at[pl.ds(0, NEG_ROWS_G)], nb.at[par],
                                  sem).wait()

        dnums = lax.GatherDimensionNumbers(
            offset_dims=(), collapsed_slice_dims=(0,), start_index_map=(0,))
        perms = [(lanes ^ s)[:, None] for s in (1, 2, 4, 8)]

        def hsum(v):
            for p in perms:
                v = v + lax.gather(
                    v, p, dimension_numbers=dnums, slice_sizes=(1,),
                    mode=lax.GatherScatterMode.PROMISE_IN_BOUNDS)
            return v

        def compute(g, par):
            cbuf = cb.at[par]
            pbuf = pb.at[par]
            nbuf = nb.at[par]

            def sample_body(i, pos_vec):
                cr = [cbuf[i, pl.ds(16 * j, 16)] for j in range(4)]
                pr = [pbuf[i, pl.ds(D + 16 * j, 16)] for j in range(4)]
                acc = (cr[0] * pr[0] + cr[1] * pr[1]) + \
                      (cr[2] * pr[2] + cr[3] * pr[3])
                pos_vec = jnp.where(lanes == i, hsum(acc), pos_vec)
                v0 = jnp.zeros((16,), jnp.float32)
                v1 = jnp.zeros((16,), jnp.float32)
                for k in range(K_NEG):
                    r = i * K_NEG + k
                    nr = [nbuf[r, pl.ds(D + 16 * j, 16)] for j in range(4)]
                    na = (cr[0] * nr[0] + cr[1] * nr[1]) + \
                         (cr[2] * nr[2] + cr[3] * nr[3])
                    t = hsum(na)
                    if k < 16:
                        v0 = jnp.where(lanes == k, t, v0)
                    else:
                        v1 = jnp.where(lanes == (k - 16), t, v1)
                ns[pl.ds(i * KP, 16)] = v0
                ns[pl.ds(i * KP + 16, 16)] = v1
                return pos_vec

            pos_vec = lax.fori_loop(0, G, sample_body,
                                    jnp.zeros((16,), jnp.float32))
            ps[pl.ds(g * G, 16)] = pos_vec
            base = (wid * SPW + g * G) * KP
            pltpu.sync_copy(ns, neg_out.at[pl.ds(base, G * KP)])

        issue(0, 0)

        def group_pair(gp, carry):
            for b in range(2):
                g = gp * 2 + b

                @pl.when(g + 1 < NG)
                def _():
                    issue(g + 1, 1 - b)

                drain(b)
                compute(g, b)
            return carry

        lax.fori_loop(0, NG // 2, group_pair, 0)
        pltpu.sync_copy(ps, pos_out.at[pl.ds(wid * SPW, SPW)])

    return body(center2, pos2, neg2, cat)


def _loss_body(pos_ref, neg_ref, out_ref):
    x = pos_ref[...]
    ls_pos = jnp.minimum(x, 0.0) - jnp.log1p(jnp.exp(-jnp.abs(x)))
    y = neg_ref[...]
    col = lax.broadcasted_iota(jnp.int32, y.shape, 1)
    valid = (col % KP) < K_NEG
    ls_neg = jnp.where(valid,
                       jnp.minimum(-y, 0.0) - jnp.log1p(jnp.exp(-jnp.abs(y))),
                       0.0)
    total = -(jnp.sum(ls_pos) + jnp.sum(ls_neg))
    out_ref[...] = jnp.reshape(total, (1, 1))


def kernel(center_words, pos_context, neg_context, embedding_v, embedding_u):
    cat = _repack(embedding_v, embedding_u)
    center2 = center_words.reshape(B // G, G)
    pos2 = pos_context.reshape(B // G, G)
    neg2 = neg_context.reshape(B * K_NEG // 64, 64)
    pos_logits, neg_logits = _sc_logits(center2, pos2, neg2, cat)
    loss2 = pl.pallas_call(
        _loss_body,
        out_shape=jax.ShapeDtypeStruct((1, 1), jnp.float32),
    )(pos_logits.reshape(128, 128), neg_logits.reshape(B * KP // 128, 128))
    return loss2[0, 0]


# speedup vs baseline: 5.3547x; 1.0645x over previous
import functools

import jax
import jax.numpy as jnp
from jax import lax
from jax.experimental import pallas as pl
from jax.experimental.pallas import tpu as pltpu
from jax.experimental.pallas import tpu_sc as plsc

B = 16384
K_NEG = 20
KP = 32
D = 64
V = 1000000

NC = 2
NS = 16
NW = NC * NS
SPW = B // NW
G = 16
NG = SPW // G
NEG_ROWS_G = G * K_NEG
NEG_IDX_ROWS_G = NEG_ROWS_G // 64
NEG_IDX_ROWS_W = SPW * K_NEG // 64

REPACK_BN = 2048


def _repack_body(vt_ref, ut_ref, out_ref):
    ident = jnp.eye(D, dtype=jnp.float32)
    dn = (((0,), (0,)), ((), ()))
    v_t = jax.lax.dot_general(vt_ref[...], ident, dn,
                              precision=jax.lax.Precision.HIGHEST)
    u_t = jax.lax.dot_general(ut_ref[...], ident, dn,
                              precision=jax.lax.Precision.HIGHEST)
    out_ref[...] = jnp.concatenate((v_t, u_t), axis=-1)


def _repack(emb_v, emb_u):
    return pl.pallas_call(
        _repack_body,
        grid=((V + REPACK_BN - 1) // REPACK_BN,),
        in_specs=[
            pl.BlockSpec((D, REPACK_BN), lambda i: (0, i)),
            pl.BlockSpec((D, REPACK_BN), lambda i: (0, i)),
        ],
        out_specs=pl.BlockSpec((REPACK_BN, 2 * D), lambda i: (i, 0)),
        out_shape=jax.ShapeDtypeStruct((V, 2 * D), jnp.float32),
    )(emb_v.T, emb_u.T)


def _sc_logits(center2, pos2, neg2, cat):
    mesh = plsc.VectorSubcoreMesh(core_axis_name="c", subcore_axis_name="s")

    @functools.partial(
        pl.kernel,
        mesh=mesh,
        out_type=[
            jax.ShapeDtypeStruct((B,), jnp.float32),
            jax.ShapeDtypeStruct((B * KP,), jnp.float32),
        ],
        scratch_types=[
            pltpu.VMEM((NG, G), jnp.int32),
            pltpu.VMEM((NG, G), jnp.int32),
            pltpu.VMEM((NEG_IDX_ROWS_W, 64), jnp.int32),
            pltpu.VMEM((2, G, 2 * D), jnp.float32),
            pltpu.VMEM((2, G, 2 * D), jnp.float32),
            pltpu.VMEM((2, NEG_ROWS_G, 2 * D), jnp.float32),
            pltpu.VMEM((SPW,), jnp.float32),
            pltpu.VMEM((G * KP,), jnp.float32),
            pltpu.SemaphoreType.DMA,
            pltpu.SemaphoreType.DMA,
        ],
    )
    def body(center_r, pos_r, neg_r, cat_r, pos_out, neg_out,
             idx_c, idx_p, idx_n, cb, pb, nb, ps, ns, sem0, sem1):
        wid = lax.axis_index("s") * NC + lax.axis_index("c")
        sems = (sem0, sem1)
        lanes = lax.iota(jnp.int32, 16)

        pltpu.sync_copy(center_r.at[pl.ds(wid * NG, NG)], idx_c)
        pltpu.sync_copy(pos_r.at[pl.ds(wid * NG, NG)], idx_p)
        pltpu.sync_copy(neg_r.at[pl.ds(wid * NEG_IDX_ROWS_W, NEG_IDX_ROWS_W)],
                        idx_n)

        def issue(g, par):
            sem = sems[par]
            pltpu.async_copy(cat_r.at[idx_c.at[g]], cb.at[par], sem)
            pltpu.async_copy(cat_r.at[idx_p.at[g]], pb.at[par], sem)
            for j in range(NEG_IDX_ROWS_G):
                pltpu.async_copy(
                    cat_r.at[idx_n.at[g * NEG_IDX_ROWS_G + j]],
                    nb.at[par].at[pl.ds(j * 64, 64)], sem)

        def drain(par):
            sem = sems[par]
            pltpu.make_async_copy(cat_r.at[pl.ds(0, G)], cb.at[par], sem).wait()
            pltpu.make_async_copy(cat_r.at[pl.ds(0, G)], pb.at[par], sem).wait()
            pltpu.make_async_copy(cat_r.at[pl.ds(0, NEG_ROWS_G)], nb.at[par],
                                  sem).wait()

        dnums = lax.GatherDimensionNumbers(
            offset_dims=(), collapsed_slice_dims=(0,), start_index_map=(0,))
        perms = [(lanes ^ s)[:, None] for s in (1, 2, 4, 8)]

        def hsum(v):
            for p in perms:
                v = v + lax.gather(
                    v, p, dimension_numbers=dnums, slice_sizes=(1,),
                    mode=lax.GatherScatterMode.PROMISE_IN_BOUNDS)
            return v

        def compute(g, par):
            cbuf = cb.at[par]
            pbuf = pb.at[par]
            nbuf = nb.at[par]

            def sample_body(i, pos_vec):
                cr = [cbuf[i, pl.ds(16 * j, 16)] for j in range(4)]
                pr = [pbuf[i, pl.ds(D + 16 * j, 16)] for j in range(4)]
                acc = (cr[0] * pr[0] + cr[1] * pr[1]) + \
                      (cr[2] * pr[2] + cr[3] * pr[3])
                pos_vec = jnp.where(lanes == i, hsum(acc), pos_vec)
                v0 = jnp.zeros((16,), jnp.float32)
                v1 = jnp.zeros((16,), jnp.float32)
                for k in range(K_NEG):
                    r = i * K_NEG + k
                    nr = [nbuf[r, pl.ds(D + 16 * j, 16)] for j in range(4)]
                    na = (cr[0] * nr[0] + cr[1] * nr[1]) + \
                         (cr[2] * nr[2] + cr[3] * nr[3])
                    t = hsum(na)
                    if k < 16:
                        v0 = jnp.where(lanes == k, t, v0)
                    else:
                        v1 = jnp.where(lanes == (k - 16), t, v1)
                ns[pl.ds(i * KP, 16)] = v0
                ns[pl.ds(i * KP + 16, 16)] = v1
                return pos_vec

            pos_vec = lax.fori_loop(0, G, sample_body,
                                    jnp.zeros((16,), jnp.float32))
            ps[pl.ds(g * G, 16)] = pos_vec
            base = (wid * SPW + g * G) * KP
            pltpu.sync_copy(ns, neg_out.at[pl.ds(base, G * KP)])

        issue(0, 0)

        def group_pair(gp, carry):
            for b in range(2):
                g = gp * 2 + b

                @pl.when(g + 1 < NG)
                def _():
                    issue(g + 1, 1 - b)

                drain(b)
                compute(g, b)
            return carry

        lax.fori_loop(0, NG // 2, group_pair, 0)
        pltpu.sync_copy(ps, pos_out.at[pl.ds(wid * SPW, SPW)])

    return body(center2, pos2, neg2, cat)


def _loss_body(pos_ref, neg_ref, out_ref):
    x = pos_ref[...]
    ls_pos = jnp.minimum(x, 0.0) - jnp.log1p(jnp.exp(-jnp.abs(x)))
    y = neg_ref[...]
    col = lax.broadcasted_iota(jnp.int32, y.shape, 1)
    valid = (col % KP) < K_NEG
    ls_neg = jnp.where(valid,
                       jnp.minimum(-y, 0.0) - jnp.log1p(jnp.exp(-jnp.abs(y))),
                       0.0)
    total = -(jnp.sum(ls_pos) + jnp.sum(ls_neg))
    out_ref[...] = jnp.reshape(total, (1, 1))


def kernel(center_words, pos_context, neg_context, embedding_v, embedding_u):
    cat = _repack(embedding_v, embedding_u)
    center2 = center_words.reshape(B // G, G)
    pos2 = pos_context.reshape(B // G, G)
    neg2 = neg_context.reshape(B * K_NEG // 64, 64)
    pos_logits, neg_logits = _sc_logits(center2, pos2, neg2, cat)
    loss2 = pl.pallas_call(
        _loss_body,
        out_shape=jax.ShapeDtypeStruct((1, 1), jnp.float32),
    )(pos_logits.reshape(128, 128), neg_logits.reshape(B * KP // 128, 128))
    return loss2[0, 0]


# speedup vs baseline: 8.2774x; 1.5458x over previous
import functools

import jax
import jax.numpy as jnp
from jax import lax
from jax.experimental import pallas as pl
from jax.experimental.pallas import tpu as pltpu
from jax.experimental.pallas import tpu_sc as plsc

B = 16384
K_NEG = 20
KP = 32
D = 64
V = 1000000

NC = 2
NS = 16
NW = NC * NS
SPW = B // NW
G = 16
NG = SPW // G
NEG_ROWS_G = G * K_NEG
NEG_IDX_ROWS_G = NEG_ROWS_G // 64
NEG_IDX_ROWS_W = SPW * K_NEG // 64

REPACK_BN = 2048


def _repack_body(vt_ref, ut_ref, out_ref):
    out_ref[...] = jnp.concatenate(
        (vt_ref[...].T, ut_ref[...].T), axis=-1)


def _repack(emb_v, emb_u):
    return pl.pallas_call(
        _repack_body,
        grid=((V + REPACK_BN - 1) // REPACK_BN,),
        in_specs=[
            pl.BlockSpec((D, REPACK_BN), lambda i: (0, i)),
            pl.BlockSpec((D, REPACK_BN), lambda i: (0, i)),
        ],
        out_specs=pl.BlockSpec((REPACK_BN, 2 * D), lambda i: (i, 0)),
        out_shape=jax.ShapeDtypeStruct((V, 2 * D), jnp.float32),
    )(emb_v.T, emb_u.T)


def _sc_logits(center2, pos2, neg2, cat):
    mesh = plsc.VectorSubcoreMesh(core_axis_name="c", subcore_axis_name="s")

    @functools.partial(
        pl.kernel,
        mesh=mesh,
        out_type=[
            jax.ShapeDtypeStruct((B,), jnp.float32),
            jax.ShapeDtypeStruct((B * KP,), jnp.float32),
        ],
        scratch_types=[
            pltpu.VMEM((NG, G), jnp.int32),
            pltpu.VMEM((NG, G), jnp.int32),
            pltpu.VMEM((NEG_IDX_ROWS_W, 64), jnp.int32),
            pltpu.VMEM((2, G, 2 * D), jnp.float32),
            pltpu.VMEM((2, G, 2 * D), jnp.float32),
            pltpu.VMEM((2, NEG_ROWS_G, 2 * D), jnp.float32),
            pltpu.VMEM((SPW,), jnp.float32),
            pltpu.VMEM((G * KP,), jnp.float32),
            pltpu.SemaphoreType.DMA,
            pltpu.SemaphoreType.DMA,
        ],
    )
    def body(center_r, pos_r, neg_r, cat_r, pos_out, neg_out,
             idx_c, idx_p, idx_n, cb, pb, nb, ps, ns, sem0, sem1):
        wid = lax.axis_index("s") * NC + lax.axis_index("c")
        sems = (sem0, sem1)
        lanes = lax.iota(jnp.int32, 16)

        pltpu.sync_copy(center_r.at[pl.ds(wid * NG, NG)], idx_c)
        pltpu.sync_copy(pos_r.at[pl.ds(wid * NG, NG)], idx_p)
        pltpu.sync_copy(neg_r.at[pl.ds(wid * NEG_IDX_ROWS_W, NEG_IDX_ROWS_W)],
                        idx_n)

        def issue(g, par):
            sem = sems[par]
            pltpu.async_copy(cat_r.at[idx_c.at[g]], cb.at[par], sem)
            pltpu.async_copy(cat_r.at[idx_p.at[g]], pb.at[par], sem)
            for j in range(NEG_IDX_ROWS_G):
                pltpu.async_copy(
                    cat_r.at[idx_n.at[g * NEG_IDX_ROWS_G + j]],
                    nb.at[par].at[pl.ds(j * 64, 64)], sem)

        def drain(par):
            sem = sems[par]
            pltpu.make_async_copy(cat_r.at[pl.ds(0, G)], cb.at[par], sem).wait()
            pltpu.make_async_copy(cat_r.at[pl.ds(0, G)], pb.at[par], sem).wait()
            pltpu.make_async_copy(cat_r.at[pl.ds(0, NEG_ROWS_G)], nb.at[par],
                                  sem).wait()

        dnums = lax.GatherDimensionNumbers(
            offset_dims=(), collapsed_slice_dims=(0,), start_index_map=(0,))
        perms = [(lanes ^ s)[:, None] for s in (1, 2, 4, 8)]

        def hsum(v):
            for p in perms:
                v = v + lax.gather(
                    v, p, dimension_numbers=dnums, slice_sizes=(1,),
                    mode=lax.GatherScatterMode.PROMISE_IN_BOUNDS)
            return v

        def compute(g, par):
            cbuf = cb.at[par]
            pbuf = pb.at[par]
            nbuf = nb.at[par]

            def sample_body(i, pos_vec):
                cr = [cbuf[i, pl.ds(16 * j, 16)] for j in range(4)]
                pr = [pbuf[i, pl.ds(D + 16 * j, 16)] for j in range(4)]
                acc = (cr[0] * pr[0] + cr[1] * pr[1]) + \
                      (cr[2] * pr[2] + cr[3] * pr[3])
                pos_vec = jnp.where(lanes == i, hsum(acc), pos_vec)
                v0 = jnp.zeros((16,), jnp.float32)
                v1 = jnp.zeros((16,), jnp.float32)
                for k in range(K_NEG):
                    r = i * K_NEG + k
                    nr = [nbuf[r, pl.ds(D + 16 * j, 16)] for j in range(4)]
                    na = (cr[0] * nr[0] + cr[1] * nr[1]) + \
                         (cr[2] * nr[2] + cr[3] * nr[3])
                    t = hsum(na)
                    if k < 16:
                        v0 = jnp.where(lanes == k, t, v0)
                    else:
                        v1 = jnp.where(lanes == (k - 16), t, v1)
                ns[pl.ds(i * KP, 16)] = v0
                ns[pl.ds(i * KP + 16, 16)] = v1
                return pos_vec

            pos_vec = lax.fori_loop(0, G, sample_body,
                                    jnp.zeros((16,), jnp.float32))
            ps[pl.ds(g * G, 16)] = pos_vec
            base = (wid * SPW + g * G) * KP
            pltpu.sync_copy(ns, neg_out.at[pl.ds(base, G * KP)])

        issue(0, 0)

        def group_pair(gp, carry):
            for b in range(2):
                g = gp * 2 + b

                @pl.when(g + 1 < NG)
                def _():
                    issue(g + 1, 1 - b)

                drain(b)
                compute(g, b)
            return carry

        lax.fori_loop(0, NG // 2, group_pair, 0)
        pltpu.sync_copy(ps, pos_out.at[pl.ds(wid * SPW, SPW)])

    return body(center2, pos2, neg2, cat)


def _loss_body(pos_ref, neg_ref, out_ref):
    x = pos_ref[...]
    ls_pos = jnp.minimum(x, 0.0) - jnp.log1p(jnp.exp(-jnp.abs(x)))
    y = neg_ref[...]
    col = lax.broadcasted_iota(jnp.int32, y.shape, 1)
    valid = (col % KP) < K_NEG
    ls_neg = jnp.where(valid,
                       jnp.minimum(-y, 0.0) - jnp.log1p(jnp.exp(-jnp.abs(y))),
                       0.0)
    total = -(jnp.sum(ls_pos) + jnp.sum(ls_neg))
    out_ref[...] = jnp.reshape(total, (1, 1))


def kernel(center_words, pos_context, neg_context, embedding_v, embedding_u):
    cat = _repack(embedding_v, embedding_u)
    center2 = center_words.reshape(B // G, G)
    pos2 = pos_context.reshape(B // G, G)
    neg2 = neg_context.reshape(B * K_NEG // 64, 64)
    pos_logits, neg_logits = _sc_logits(center2, pos2, neg2, cat)
    loss2 = pl.pallas_call(
        _loss_body,
        out_shape=jax.ShapeDtypeStruct((1, 1), jnp.float32),
    )(pos_logits.reshape(128, 128), neg_logits.reshape(B * KP // 128, 128))
    return loss2[0, 0]


# speedup vs baseline: 11.3092x; 1.3663x over previous
import functools

import jax
import jax.numpy as jnp
from jax import lax
from jax.experimental import pallas as pl
from jax.experimental.pallas import tpu as pltpu
from jax.experimental.pallas import tpu_sc as plsc

B = 16384
K_NEG = 20
KP = 32
D = 64
V = 1000000

NC = 2
NS = 16
NW = NC * NS
SPW = B // NW
G = 16
NG = SPW // G
NEG_ROWS_G = G * K_NEG
NEG_IDX_ROWS_G = NEG_ROWS_G // 64
NEG_IDX_ROWS_W = SPW * K_NEG // 64

REPACK_BN = 8192


def _repack_body(vt_ref, ut_ref, out_ref):
    out_ref[...] = jnp.concatenate(
        (vt_ref[...].T, ut_ref[...].T), axis=-1)


def _repack(emb_v, emb_u):
    return pl.pallas_call(
        _repack_body,
        grid=((V + REPACK_BN - 1) // REPACK_BN,),
        in_specs=[
            pl.BlockSpec((D, REPACK_BN), lambda i: (0, i)),
            pl.BlockSpec((D, REPACK_BN), lambda i: (0, i)),
        ],
        out_specs=pl.BlockSpec((REPACK_BN, 2 * D), lambda i: (i, 0)),
        out_shape=jax.ShapeDtypeStruct((V, 2 * D), jnp.float32),
    )(emb_v.T, emb_u.T)


def _sc_logits(center2, pos2, neg2, cat):
    mesh = plsc.VectorSubcoreMesh(core_axis_name="c", subcore_axis_name="s")

    @functools.partial(
        pl.kernel,
        mesh=mesh,
        out_type=[
            jax.ShapeDtypeStruct((B,), jnp.float32),
            jax.ShapeDtypeStruct((B * KP,), jnp.float32),
        ],
        scratch_types=[
            pltpu.VMEM((NG, G), jnp.int32),
            pltpu.VMEM((NG, G), jnp.int32),
            pltpu.VMEM((NEG_IDX_ROWS_W, 64), jnp.int32),
            pltpu.VMEM((2, G, 2 * D), jnp.float32),
            pltpu.VMEM((2, G, 2 * D), jnp.float32),
            pltpu.VMEM((2, NEG_ROWS_G, 2 * D), jnp.float32),
            pltpu.VMEM((SPW,), jnp.float32),
            pltpu.VMEM((G * KP,), jnp.float32),
            pltpu.SemaphoreType.DMA,
            pltpu.SemaphoreType.DMA,
        ],
    )
    def body(center_r, pos_r, neg_r, cat_r, pos_out, neg_out,
             idx_c, idx_p, idx_n, cb, pb, nb, ps, ns, sem0, sem1):
        wid = lax.axis_index("s") * NC + lax.axis_index("c")
        sems = (sem0, sem1)
        lanes = lax.iota(jnp.int32, 16)

        pltpu.sync_copy(center_r.at[pl.ds(wid * NG, NG)], idx_c)
        pltpu.sync_copy(pos_r.at[pl.ds(wid * NG, NG)], idx_p)
        pltpu.sync_copy(neg_r.at[pl.ds(wid * NEG_IDX_ROWS_W, NEG_IDX_ROWS_W)],
                        idx_n)

        def issue(g, par):
            sem = sems[par]
            pltpu.async_copy(cat_r.at[idx_c.at[g]], cb.at[par], sem)
            pltpu.async_copy(cat_r.at[idx_p.at[g]], pb.at[par], sem)
            for j in range(NEG_IDX_ROWS_G):
                pltpu.async_copy(
                    cat_r.at[idx_n.at[g * NEG_IDX_ROWS_G + j]],
                    nb.at[par].at[pl.ds(j * 64, 64)], sem)

        def drain(par):
            sem = sems[par]
            pltpu.make_async_copy(cat_r.at[pl.ds(0, G)], cb.at[par], sem).wait()
            pltpu.make_async_copy(cat_r.at[pl.ds(0, G)], pb.at[par], sem).wait()
            pltpu.make_async_copy(cat_r.at[pl.ds(0, NEG_ROWS_G)], nb.at[par],
                                  sem).wait()

        dnums = lax.GatherDimensionNumbers(
            offset_dims=(), collapsed_slice_dims=(0,), start_index_map=(0,))
        perms = [(lanes ^ s)[:, None] for s in (1, 2, 4, 8)]

        def hsum(v):
            for p in perms:
                v = v + lax.gather(
                    v, p, dimension_numbers=dnums, slice_sizes=(1,),
                    mode=lax.GatherScatterMode.PROMISE_IN_BOUNDS)
            return v

        def compute(g, par):
            cbuf = cb.at[par]
            pbuf = pb.at[par]
            nbuf = nb.at[par]

            def sample_body(i, pos_vec):
                cr = [cbuf[i, pl.ds(16 * j, 16)] for j in range(4)]
                pr = [pbuf[i, pl.ds(D + 16 * j, 16)] for j in range(4)]
                acc = (cr[0] * pr[0] + cr[1] * pr[1]) + \
                      (cr[2] * pr[2] + cr[3] * pr[3])
                pos_vec = jnp.where(lanes == i, hsum(acc), pos_vec)
                v0 = jnp.zeros((16,), jnp.float32)
                v1 = jnp.zeros((16,), jnp.float32)
                for k in range(K_NEG):
                    r = i * K_NEG + k
                    nr = [nbuf[r, pl.ds(D + 16 * j, 16)] for j in range(4)]
                    na = (cr[0] * nr[0] + cr[1] * nr[1]) + \
                         (cr[2] * nr[2] + cr[3] * nr[3])
                    t = hsum(na)
                    if k < 16:
                        v0 = jnp.where(lanes == k, t, v0)
                    else:
                        v1 = jnp.where(lanes == (k - 16), t, v1)
                ns[pl.ds(i * KP, 16)] = v0
                ns[pl.ds(i * KP + 16, 16)] = v1
                return pos_vec

            pos_vec = lax.fori_loop(0, G, sample_body,
                                    jnp.zeros((16,), jnp.float32))
            ps[pl.ds(g * G, 16)] = pos_vec
            base = (wid * SPW + g * G) * KP
            pltpu.sync_copy(ns, neg_out.at[pl.ds(base, G * KP)])

        issue(0, 0)

        def group_pair(gp, carry):
            for b in range(2):
                g = gp * 2 + b

                @pl.when(g + 1 < NG)
                def _():
                    issue(g + 1, 1 - b)

                drain(b)
                compute(g, b)
            return carry

        lax.fori_loop(0, NG // 2, group_pair, 0)
        pltpu.sync_copy(ps, pos_out.at[pl.ds(wid * SPW, SPW)])

    return body(center2, pos2, neg2, cat)


def _loss_body(pos_ref, neg_ref, out_ref):
    x = pos_ref[...]
    ls_pos = jnp.minimum(x, 0.0) - jnp.log1p(jnp.exp(-jnp.abs(x)))
    y = neg_ref[...]
    col = lax.broadcasted_iota(jnp.int32, y.shape, 1)
    valid = (col % KP) < K_NEG
    ls_neg = jnp.where(valid,
                       jnp.minimum(-y, 0.0) - jnp.log1p(jnp.exp(-jnp.abs(y))),
                       0.0)
    total = -(jnp.sum(ls_pos) + jnp.sum(ls_neg))
    out_ref[...] = jnp.reshape(total, (1, 1))


def kernel(center_words, pos_context, neg_context, embedding_v, embedding_u):
    cat = _repack(embedding_v, embedding_u)
    center2 = center_words.reshape(B // G, G)
    pos2 = pos_context.reshape(B // G, G)
    neg2 = neg_context.reshape(B * K_NEG // 64, 64)
    pos_logits, neg_logits = _sc_logits(center2, pos2, neg2, cat)
    loss2 = pl.pallas_call(
        _loss_body,
        out_shape=jax.ShapeDtypeStruct((1, 1), jnp.float32),
    )(pos_logits.reshape(128, 128), neg_logits.reshape(B * KP // 128, 128))
    return loss2[0, 0]


# speedup vs baseline: 11.9504x; 1.0567x over previous
import functools

import jax
import jax.numpy as jnp
from jax import lax
from jax.experimental import pallas as pl
from jax.experimental.pallas import tpu as pltpu
from jax.experimental.pallas import tpu_sc as plsc

B = 16384
K_NEG = 20
KP = 32
D = 64
V = 1000000

NC = 2
NS = 16
NW = NC * NS
SPW = B // NW
G = 16
NG = SPW // G
NEG_ROWS_G = G * K_NEG
NEG_IDX_ROWS_G = NEG_ROWS_G // 64
NEG_IDX_ROWS_W = SPW * K_NEG // 64

REPACK_BN = 16384


def _repack_body(vt_ref, ut_ref, out_ref):
    out_ref[...] = jnp.concatenate(
        (vt_ref[...].T, ut_ref[...].T), axis=-1)


def _repack(emb_v, emb_u):
    return pl.pallas_call(
        _repack_body,
        grid=((V + REPACK_BN - 1) // REPACK_BN,),
        in_specs=[
            pl.BlockSpec((D, REPACK_BN), lambda i: (0, i)),
            pl.BlockSpec((D, REPACK_BN), lambda i: (0, i)),
        ],
        out_specs=pl.BlockSpec((REPACK_BN, 2 * D), lambda i: (i, 0)),
        out_shape=jax.ShapeDtypeStruct((V, 2 * D), jnp.float32),
    )(emb_v.T, emb_u.T)


def _sc_logits(center2, pos2, neg2, cat):
    mesh = plsc.VectorSubcoreMesh(core_axis_name="c", subcore_axis_name="s")

    @functools.partial(
        pl.kernel,
        mesh=mesh,
        out_type=[
            jax.ShapeDtypeStruct((B,), jnp.float32),
            jax.ShapeDtypeStruct((B * KP,), jnp.float32),
        ],
        scratch_types=[
            pltpu.VMEM((NG, G), jnp.int32),
            pltpu.VMEM((NG, G), jnp.int32),
            pltpu.VMEM((NEG_IDX_ROWS_W, 64), jnp.int32),
            pltpu.VMEM((2, G, 2 * D), jnp.float32),
            pltpu.VMEM((2, G, 2 * D), jnp.float32),
            pltpu.VMEM((2, NEG_ROWS_G, 2 * D), jnp.float32),
            pltpu.VMEM((SPW,), jnp.float32),
            pltpu.VMEM((G * KP,), jnp.float32),
            pltpu.SemaphoreType.DMA,
            pltpu.SemaphoreType.DMA,
        ],
    )
    def body(center_r, pos_r, neg_r, cat_r, pos_out, neg_out,
             idx_c, idx_p, idx_n, cb, pb, nb, ps, ns, sem0, sem1):
        wid = lax.axis_index("s") * NC + lax.axis_index("c")
        sems = (sem0, sem1)
        lanes = lax.iota(jnp.int32, 16)

        pltpu.sync_copy(center_r.at[pl.ds(wid * NG, NG)], idx_c)
        pltpu.sync_copy(pos_r.at[pl.ds(wid * NG, NG)], idx_p)
        pltpu.sync_copy(neg_r.at[pl.ds(wid * NEG_IDX_ROWS_W, NEG_IDX_ROWS_W)],
                        idx_n)

        def issue(g, par):
            sem = sems[par]
            pltpu.async_copy(cat_r.at[idx_c.at[g]], cb.at[par], sem)
            pltpu.async_copy(cat_r.at[idx_p.at[g]], pb.at[par], sem)
            for j in range(NEG_IDX_ROWS_G):
                pltpu.async_copy(
                    cat_r.at[idx_n.at[g * NEG_IDX_ROWS_G + j]],
                    nb.at[par].at[pl.ds(j * 64, 64)], sem)

        def drain(par):
            sem = sems[par]
            pltpu.make_async_copy(cat_r.at[pl.ds(0, G)], cb.at[par], sem).wait()
            pltpu.make_async_copy(cat_r.at[pl.ds(0, G)], pb.at[par], sem).wait()
            pltpu.make_async_copy(cat_r.at[pl.ds(0, NEG_ROWS_G)], nb.at[par],
                                  sem).wait()

        dnums = lax.GatherDimensionNumbers(
            offset_dims=(), collapsed_slice_dims=(0,), start_index_map=(0,))
        perms = [(lanes ^ s)[:, None] for s in (1, 2, 4, 8)]

        def hsum(v):
            for p in perms:
                v = v + lax.gather(
                    v, p, dimension_numbers=dnums, slice_sizes=(1,),
                    mode=lax.GatherScatterMode.PROMISE_IN_BOUNDS)
            return v

        def compute(g, par):
            cbuf = cb.at[par]
            pbuf = pb.at[par]
            nbuf = nb.at[par]

            def sample_body(i, pos_vec):
                cr = [cbuf[i, pl.ds(16 * j, 16)] for j in range(4)]
                pr = [pbuf[i, pl.ds(D + 16 * j, 16)] for j in range(4)]
                acc = (cr[0] * pr[0] + cr[1] * pr[1]) + \
                      (cr[2] * pr[2] + cr[3] * pr[3])
                pos_vec = jnp.where(lanes == i, hsum(acc), pos_vec)
                v0 = jnp.zeros((16,), jnp.float32)
                v1 = jnp.zeros((16,), jnp.float32)
                for k in range(K_NEG):
                    r = i * K_NEG + k
                    nr = [nbuf[r, pl.ds(D + 16 * j, 16)] for j in range(4)]
                    na = (cr[0] * nr[0] + cr[1] * nr[1]) + \
                         (cr[2] * nr[2] + cr[3] * nr[3])
                    t = hsum(na)
                    if k < 16:
                        v0 = jnp.where(lanes == k, t, v0)
                    else:
                        v1 = jnp.where(lanes == (k - 16), t, v1)
                ns[pl.ds(i * KP, 16)] = v0
                ns[pl.ds(i * KP + 16, 16)] = v1
                return pos_vec

            pos_vec = lax.fori_loop(0, G, sample_body,
                                    jnp.zeros((16,), jnp.float32))
            ps[pl.ds(g * G, 16)] = pos_vec
            base = (wid * SPW + g * G) * KP
            pltpu.sync_copy(ns, neg_out.at[pl.ds(base, G * KP)])

        issue(0, 0)

        def group_pair(gp, carry):
            for b in range(2):
                g = gp * 2 + b

                @pl.when(g + 1 < NG)
                def _():
                    issue(g + 1, 1 - b)

                drain(b)
                compute(g, b)
            return carry

        lax.fori_loop(0, NG // 2, group_pair, 0)
        pltpu.sync_copy(ps, pos_out.at[pl.ds(wid * SPW, SPW)])

    return body(center2, pos2, neg2, cat)


def _loss_body(pos_ref, neg_ref, out_ref):
    x = pos_ref[...]
    ls_pos = jnp.minimum(x, 0.0) - jnp.log1p(jnp.exp(-jnp.abs(x)))
    y = neg_ref[...]
    col = lax.broadcasted_iota(jnp.int32, y.shape, 1)
    valid = (col % KP) < K_NEG
    ls_neg = jnp.where(valid,
                       jnp.minimum(-y, 0.0) - jnp.log1p(jnp.exp(-jnp.abs(y))),
                       0.0)
    total = -(jnp.sum(ls_pos) + jnp.sum(ls_neg))
    out_ref[...] = jnp.reshape(total, (1, 1))


def kernel(center_words, pos_context, neg_context, embedding_v, embedding_u):
    cat = _repack(embedding_v, embedding_u)
    center2 = center_words.reshape(B // G, G)
    pos2 = pos_context.reshape(B // G, G)
    neg2 = neg_context.reshape(B * K_NEG // 64, 64)
    pos_logits, neg_logits = _sc_logits(center2, pos2, neg2, cat)
    loss2 = pl.pallas_call(
        _loss_body,
        out_shape=jax.ShapeDtypeStruct((1, 1), jnp.float32),
    )(pos_logits.reshape(128, 128), neg_logits.reshape(B * KP // 128, 128))
    return loss2[0, 0]
